# Initial kernel scaffold; baseline (speedup 1.0000x reference)
import jax
import jax.numpy as jnp
from jax import lax
from jax.experimental import pallas as pl
from jax.experimental.pallas import tpu as pltpu
from jax.experimental.pallas import tpu_sc as plsc

N_NODES = 10000
N_EDGES = 320000
D = 128
N_HOPS = 3

NC = 2
NS = 16
NW = NC * NS
L = 16

CHUNK = 128
EP = ((N_EDGES + NW * CHUNK - 1) // (NW * CHUNK)) * (NW * CHUNK)
EPW = EP // NW
NCH = EPW // CHUNK
RPT = N_NODES // NS
ZR = 125


def _hop_body(agg_hbm, row_hbm, col_hbm, tr_hbm, out_hbm,
              acc, zbuf, row_v, col_v, tr_v, gat_v, sem):
    c = lax.axis_index("c")
    s = lax.axis_index("s")
    wid = c * NS + s
    base = wid * EPW

    @pl.loop(0, ZR)
    def _(r):
        for k in range(D // L):
            zbuf[r, pl.ds(k * L, L)] = jnp.zeros((L,), jnp.float32)

    for j in range(RPT // ZR):
        pltpu.sync_copy(zbuf, acc.at[pl.ds(s * RPT + j * ZR, ZR)])
    plsc.subcore_barrier()

    @pl.loop(0, NCH)
    def _(ch):
        ebase = base + ch * CHUNK
        pltpu.sync_copy(row_hbm.at[pl.ds(ebase, CHUNK)], row_v)
        pltpu.sync_copy(col_hbm.at[pl.ds(ebase, CHUNK)], col_v)
        pltpu.sync_copy(tr_hbm.at[pl.ds(ebase, CHUNK)], tr_v)
        pltpu.async_copy(agg_hbm.at[row_v], gat_v, sem).wait()

        @pl.loop(0, CHUNK)
        def _(e):
            t = tr_v[e]
            for d in range(D // L):
                sl = pl.ds(d * L, L)
                gat_v[e, sl] = gat_v[e, sl] * t

        pltpu.sync_copy(gat_v, acc.at[col_v], add=True)

    plsc.subcore_barrier()
    for j in range(RPT // ZR):
        rs = s * RPT + j * ZR
        pltpu.sync_copy(acc.at[pl.ds(rs, ZR)], out_hbm.at[c, pl.ds(rs, ZR)])


_hop = pl.kernel(
    _hop_body,
    out_type=jax.ShapeDtypeStruct((NC, N_NODES, D), jnp.float32),
    mesh=plsc.VectorSubcoreMesh(core_axis_name="c", subcore_axis_name="s"),
    scratch_types=[
        pltpu.VMEM_SHARED((N_NODES, D), jnp.float32),
        pltpu.VMEM((ZR, D), jnp.float32),
        pltpu.VMEM((CHUNK,), jnp.int32),
        pltpu.VMEM((CHUNK,), jnp.int32),
        pltpu.VMEM((CHUNK, L), jnp.float32),
        pltpu.VMEM((CHUNK, D), jnp.float32),
        pltpu.SemaphoreType.DMA,
    ],
)


def _add_body(a_ref, b_ref, o_ref):
    o_ref[...] = a_ref[0] + b_ref[0]


_combine = pl.pallas_call(
    _add_body,
    grid=(20,),
    in_specs=[
        pl.BlockSpec((1, N_NODES // 20, D), lambda i: (0, i, 0)),
        pl.BlockSpec((1, N_NODES // 20, D), lambda i: (1, i, 0)),
    ],
    out_specs=pl.BlockSpec((N_NODES // 20, D), lambda i: (i, 0)),
    out_shape=jax.ShapeDtypeStruct((N_NODES, D), jnp.float32),
)


@jax.jit
def kernel(embed, edge_index, trend):
    row = edge_index[0].astype(jnp.int32)
    col = edge_index[1].astype(jnp.int32)
    pad = EP - N_EDGES
    row = jnp.concatenate([row, jnp.zeros((pad,), jnp.int32)])
    col = jnp.concatenate([col, jnp.zeros((pad,), jnp.int32)])
    tr = jnp.concatenate([trend, jnp.zeros((pad,), jnp.float32)])
    tr16 = jnp.broadcast_to(tr[:, None], (EP, L))

    agg = embed
    embs = [embed]
    for _ in range(N_HOPS):
        partials = _hop(agg, row, col, tr16)
        agg = _combine(partials, partials)
        embs.append(agg)
    return jnp.stack(embs, axis=1)


# speedup vs baseline: 2.7362x; 2.7362x over previous
import jax
import jax.numpy as jnp
from jax import lax
from jax.experimental import pallas as pl
from jax.experimental.pallas import tpu as pltpu
from jax.experimental.pallas import tpu_sc as plsc

N_NODES = 10000
N_EDGES = 320000
D = 128
N_HOPS = 3

NC = 2
NS = 16
NW = NC * NS
L = 16

CHUNK = 128
EP = ((N_EDGES + NW * CHUNK - 1) // (NW * CHUNK)) * (NW * CHUNK)
EPW = EP // NW
NCH = EPW // CHUNK
NP = 10240
RPT = NP // NS
ZR = 128


def _hop_body(agg_hbm, row_hbm, col_hbm, tr_hbm, out_hbm,
              acc, row_v, col_v, tr_v, gat_v, sem):
    c = lax.axis_index("c")
    s = lax.axis_index("s")
    wid = c * NS + s
    base = wid * EPW

    @pl.loop(0, ZR)
    def _(r):
        for k in range(D // L):
            gat_v[r, pl.ds(k * L, L)] = jnp.zeros((L,), jnp.float32)

    for j in range(RPT // ZR):
        pltpu.sync_copy(gat_v, acc.at[pl.ds(s * RPT + j * ZR, ZR)])
    plsc.subcore_barrier()

    @pl.loop(0, NCH)
    def _(ch):
        ebase = base + ch * CHUNK
        pltpu.sync_copy(row_hbm.at[pl.ds(ebase, CHUNK)], row_v)
        pltpu.sync_copy(col_hbm.at[pl.ds(ebase, CHUNK)], col_v)
        pltpu.sync_copy(tr_hbm.at[pl.ds(ebase, CHUNK)], tr_v)
        pltpu.async_copy(agg_hbm.at[row_v], gat_v, sem).wait()

        @pl.loop(0, CHUNK)
        def _(e):
            t = tr_v[e]
            for d in range(D // L):
                sl = pl.ds(d * L, L)
                gat_v[e, sl] = gat_v[e, sl] * t

        pltpu.sync_copy(gat_v, acc.at[col_v], add=True)

    plsc.subcore_barrier()
    for j in range(RPT // ZR):
        rs = s * RPT + j * ZR
        pltpu.sync_copy(acc.at[pl.ds(rs, ZR)], out_hbm.at[c, pl.ds(rs, ZR)])


_hop = pl.kernel(
    _hop_body,
    out_type=jax.ShapeDtypeStruct((NC, NP, D), jnp.float32),
    mesh=plsc.VectorSubcoreMesh(core_axis_name="c", subcore_axis_name="s"),
    scratch_types=[
        pltpu.VMEM_SHARED((NP, D), jnp.float32),
        pltpu.VMEM((CHUNK,), jnp.int32),
        pltpu.VMEM((CHUNK,), jnp.int32),
        pltpu.VMEM((CHUNK, L), jnp.float32),
        pltpu.VMEM((CHUNK, D), jnp.float32),
        pltpu.SemaphoreType.DMA,
    ],
)


def _add_body(a_ref, b_ref, o_ref):
    o_ref[...] = a_ref[0] + b_ref[0]


_combine = pl.pallas_call(
    _add_body,
    grid=(5,),
    in_specs=[
        pl.BlockSpec((1, NP // 5, D), lambda i: (0, i, 0)),
        pl.BlockSpec((1, NP // 5, D), lambda i: (1, i, 0)),
    ],
    out_specs=pl.BlockSpec((NP // 5, D), lambda i: (i, 0)),
    out_shape=jax.ShapeDtypeStruct((NP, D), jnp.float32),
)


@jax.jit
def kernel(embed, edge_index, trend):
    row = edge_index[0].astype(jnp.int32)
    col = edge_index[1].astype(jnp.int32)
    pad = EP - N_EDGES
    row = jnp.concatenate([row, jnp.zeros((pad,), jnp.int32)])
    col = jnp.concatenate([col, jnp.zeros((pad,), jnp.int32)])
    tr = jnp.concatenate([trend, jnp.zeros((pad,), jnp.float32)])
    tr16 = jnp.broadcast_to(tr[:, None], (EP, L))

    agg = jnp.concatenate([embed, jnp.zeros((NP - N_NODES, D), jnp.float32)])
    embs = [embed]
    for _ in range(N_HOPS):
        partials = _hop(agg, row, col, tr16)
        agg = _combine(partials, partials)
        embs.append(agg[:N_NODES])
    return jnp.stack(embs, axis=1)


# speedup vs baseline: 2.7805x; 1.0162x over previous
import jax
import jax.numpy as jnp
from jax import lax
from jax.experimental import pallas as pl
from jax.experimental.pallas import tpu as pltpu
from jax.experimental.pallas import tpu_sc as plsc

N_NODES = 10000
N_EDGES = 320000
D = 128
N_HOPS = 3

NC = 2
NS = 16
NW = NC * NS
L = 16

CHUNK = 128
NCH = 80
EPW = NCH * CHUNK
EP = NW * EPW
NP = 10112
RPT = NP // NS


def _hop_body(agg_hbm, row_hbm, col_hbm, tr_hbm, out_hbm,
              acc, row_v, col_v, tr_v, gat_v, sem_i, sem_g):
    c = lax.axis_index("c")
    s = lax.axis_index("s")
    wid = c * NS + s
    base = wid * EPW

    def idx_start(ch, b):
        ebase = base + ch * CHUNK
        pltpu.async_copy(row_hbm.at[pl.ds(ebase, CHUNK)], row_v.at[b], sem_i.at[b])
        pltpu.async_copy(col_hbm.at[pl.ds(ebase, CHUNK)], col_v.at[b], sem_i.at[b])
        pltpu.async_copy(tr_hbm.at[pl.ds(ebase, CHUNK)], tr_v.at[b], sem_i.at[b])

    def idx_wait(ch, b):
        ebase = base + ch * CHUNK
        pltpu.make_async_copy(row_hbm.at[pl.ds(ebase, CHUNK)], row_v.at[b], sem_i.at[b]).wait()
        pltpu.make_async_copy(col_hbm.at[pl.ds(ebase, CHUNK)], col_v.at[b], sem_i.at[b]).wait()
        pltpu.make_async_copy(tr_hbm.at[pl.ds(ebase, CHUNK)], tr_v.at[b], sem_i.at[b]).wait()

    def gat_start(b, g):
        pltpu.async_copy(agg_hbm.at[row_v.at[b]], gat_v.at[g], sem_g.at[g])

    def gat_wait(b, g):
        pltpu.make_async_copy(agg_hbm.at[row_v.at[b]], gat_v.at[g], sem_g.at[g]).wait()

    def scale_scatter(b, gb):
        @pl.loop(0, CHUNK // L)
        def _(g):
            t16 = tr_v[b, pl.ds(g * L, L)]
            for l in range(L):
                e = g * L + l
                t = t16[l]
                for d in range(D // L):
                    sl = pl.ds(d * L, L)
                    gat_v[gb, e, sl] = gat_v[gb, e, sl] * t

        pltpu.sync_copy(gat_v.at[gb], acc.at[col_v.at[b]], add=True)

    @pl.loop(0, CHUNK)
    def _(r):
        for k in range(D // L):
            gat_v[0, r, pl.ds(k * L, L)] = jnp.zeros((L,), jnp.float32)

    nz = RPT // CHUNK
    rem = RPT - nz * CHUNK
    for j in range(nz):
        pltpu.sync_copy(gat_v.at[0], acc.at[pl.ds(s * RPT + j * CHUNK, CHUNK)])
    pltpu.sync_copy(gat_v.at[0, pl.ds(0, rem)],
                    acc.at[pl.ds(s * RPT + nz * CHUNK, rem)])
    plsc.subcore_barrier()

    for p in range(4):
        idx_start(p, p)
    idx_wait(0, 0)
    gat_start(0, 0)

    @pl.loop(0, NCH)
    def _(ch):
        ib = lax.rem(ch, 4)
        gb = lax.rem(ch, 2)

        @pl.when(ch + 1 < NCH)
        def _():
            idx_wait(ch + 1, lax.rem(ch + 1, 4))
            gat_start(lax.rem(ch + 1, 4), 1 - gb)

        gat_wait(ib, gb)
        scale_scatter(ib, gb)

        @pl.when(ch + 4 < NCH)
        def _():
            idx_start(ch + 4, ib)

    plsc.subcore_barrier()
    for j in range(nz):
        rs = s * RPT + j * CHUNK
        pltpu.sync_copy(acc.at[pl.ds(rs, CHUNK)], out_hbm.at[c, pl.ds(rs, CHUNK)])
    rs = s * RPT + nz * CHUNK
    pltpu.sync_copy(acc.at[pl.ds(rs, rem)], out_hbm.at[c, pl.ds(rs, rem)])


_hop = pl.kernel(
    _hop_body,
    out_type=jax.ShapeDtypeStruct((NC, NP, D), jnp.float32),
    mesh=plsc.VectorSubcoreMesh(core_axis_name="c", subcore_axis_name="s"),
    scratch_types=[
        pltpu.VMEM_SHARED((NP, D), jnp.float32),
        pltpu.VMEM((4, CHUNK), jnp.int32),
        pltpu.VMEM((4, CHUNK), jnp.int32),
        pltpu.VMEM((4, CHUNK), jnp.float32),
        pltpu.VMEM((2, CHUNK, D), jnp.float32),
        pltpu.SemaphoreType.DMA((4,)),
        pltpu.SemaphoreType.DMA((2,)),
    ],
)


def _add_body(a_ref, b_ref, o_ref):
    o_ref[...] = a_ref[0] + b_ref[0]


_combine = pl.pallas_call(
    _add_body,
    grid=(8,),
    in_specs=[
        pl.BlockSpec((1, NP // 8, D), lambda i: (0, i, 0)),
        pl.BlockSpec((1, NP // 8, D), lambda i: (1, i, 0)),
    ],
    out_specs=pl.BlockSpec((NP // 8, D), lambda i: (i, 0)),
    out_shape=jax.ShapeDtypeStruct((NP, D), jnp.float32),
)


@jax.jit
def kernel(embed, edge_index, trend):
    row = edge_index[0].astype(jnp.int32)
    col = edge_index[1].astype(jnp.int32)
    pad = EP - N_EDGES
    row = jnp.concatenate([row, jnp.zeros((pad,), jnp.int32)])
    col = jnp.concatenate([col, jnp.zeros((pad,), jnp.int32)])
    tr = jnp.concatenate([trend, jnp.zeros((pad,), jnp.float32)])

    agg = jnp.concatenate([embed, jnp.zeros((NP - N_NODES, D), jnp.float32)])
    embs = [embed]
    for _ in range(N_HOPS):
        partials = _hop(agg, row, col, tr)
        agg = _combine(partials, partials)
        embs.append(agg[:N_NODES])
    return jnp.stack(embs, axis=1)


# speedup vs baseline: 6.7668x; 2.4337x over previous
import jax
import jax.numpy as jnp
from jax import lax
from jax.experimental import pallas as pl
from jax.experimental.pallas import tpu as pltpu
from jax.experimental.pallas import tpu_sc as plsc

N_NODES = 10000
N_EDGES = 320000
D = 128
N_HOPS = 3

NC = 2
NS = 16
L = 16
DH = D // NC

CHUNK = 128
NCH = 160
EPT = NCH * CHUNK
EP = NS * EPT
NP = 10112
RPT = NP // NS


def _hop_body(agg_hbm, row_hbm, col_hbm, tr_hbm, out_hbm,
              table, acc, row_v, col_v, tr_v, gat_v, sem_i, sem_g):
    c = lax.axis_index("c")
    s = lax.axis_index("s")
    base = s * EPT

    def idx_start(ch, b):
        ebase = base + ch * CHUNK
        pltpu.async_copy(row_hbm.at[pl.ds(ebase, CHUNK)], row_v.at[b], sem_i.at[b])
        pltpu.async_copy(col_hbm.at[pl.ds(ebase, CHUNK)], col_v.at[b], sem_i.at[b])
        pltpu.async_copy(tr_hbm.at[pl.ds(ebase, CHUNK)], tr_v.at[b], sem_i.at[b])

    def idx_wait(ch, b):
        ebase = base + ch * CHUNK
        pltpu.make_async_copy(row_hbm.at[pl.ds(ebase, CHUNK)], row_v.at[b], sem_i.at[b]).wait()
        pltpu.make_async_copy(col_hbm.at[pl.ds(ebase, CHUNK)], col_v.at[b], sem_i.at[b]).wait()
        pltpu.make_async_copy(tr_hbm.at[pl.ds(ebase, CHUNK)], tr_v.at[b], sem_i.at[b]).wait()

    def gat_start(b, g):
        pltpu.async_copy(table.at[row_v.at[b]], gat_v.at[g], sem_g.at[g])

    def gat_wait(b, g):
        pltpu.make_async_copy(table.at[row_v.at[b]], gat_v.at[g], sem_g.at[g]).wait()

    def scale_scatter(b, gb):
        @pl.loop(0, CHUNK // L)
        def _(g):
            t16 = tr_v[b, pl.ds(g * L, L)]
            for l in range(L):
                e = g * L + l
                t = t16[l]
                for d in range(DH // L):
                    sl = pl.ds(d * L, L)
                    gat_v[gb, e, sl] = gat_v[gb, e, sl] * t

        pltpu.sync_copy(gat_v.at[gb], acc.at[col_v.at[b]], add=True)

    pltpu.sync_copy(agg_hbm.at[c, pl.ds(s * RPT, RPT)], table.at[pl.ds(s * RPT, RPT)])

    @pl.loop(0, CHUNK)
    def _(r):
        for k in range(DH // L):
            gat_v[0, r, pl.ds(k * L, L)] = jnp.zeros((L,), jnp.float32)

    nz = RPT // CHUNK
    rem = RPT - nz * CHUNK
    for j in range(nz):
        pltpu.sync_copy(gat_v.at[0], acc.at[pl.ds(s * RPT + j * CHUNK, CHUNK)])
    pltpu.sync_copy(gat_v.at[0, pl.ds(0, rem)],
                    acc.at[pl.ds(s * RPT + nz * CHUNK, rem)])
    plsc.subcore_barrier()

    for p in range(4):
        idx_start(p, p)
    idx_wait(0, 0)
    gat_start(0, 0)

    @pl.loop(0, NCH)
    def _(ch):
        ib = lax.rem(ch, 4)
        gb = lax.rem(ch, 2)

        @pl.when(ch + 1 < NCH)
        def _():
            idx_wait(ch + 1, lax.rem(ch + 1, 4))
            gat_start(lax.rem(ch + 1, 4), 1 - gb)

        gat_wait(ib, gb)
        scale_scatter(ib, gb)

        @pl.when(ch + 4 < NCH)
        def _():
            idx_start(ch + 4, ib)

    plsc.subcore_barrier()
    for j in range(nz):
        rs = s * RPT + j * CHUNK
        pltpu.sync_copy(acc.at[pl.ds(rs, CHUNK)], out_hbm.at[c, pl.ds(rs, CHUNK)])
    rs = s * RPT + nz * CHUNK
    pltpu.sync_copy(acc.at[pl.ds(rs, rem)], out_hbm.at[c, pl.ds(rs, rem)])


_hop = pl.kernel(
    _hop_body,
    out_type=jax.ShapeDtypeStruct((NC, NP, DH), jnp.float32),
    mesh=plsc.VectorSubcoreMesh(core_axis_name="c", subcore_axis_name="s"),
    scratch_types=[
        pltpu.VMEM_SHARED((NP, DH), jnp.float32),
        pltpu.VMEM_SHARED((NP, DH), jnp.float32),
        pltpu.VMEM((4, CHUNK), jnp.int32),
        pltpu.VMEM((4, CHUNK), jnp.int32),
        pltpu.VMEM((4, CHUNK), jnp.float32),
        pltpu.VMEM((2, CHUNK, DH), jnp.float32),
        pltpu.SemaphoreType.DMA((4,)),
        pltpu.SemaphoreType.DMA((2,)),
    ],
)


@jax.jit
def kernel(embed, edge_index, trend):
    row = edge_index[0].astype(jnp.int32)
    col = edge_index[1].astype(jnp.int32)
    pad = EP - N_EDGES
    row = jnp.concatenate([row, jnp.zeros((pad,), jnp.int32)])
    col = jnp.concatenate([col, jnp.zeros((pad,), jnp.int32)])
    tr = jnp.concatenate([trend, jnp.zeros((pad,), jnp.float32)])

    npad = NP - N_NODES
    agg2 = jnp.stack([
        jnp.concatenate([embed[:, :DH], jnp.zeros((npad, DH), jnp.float32)]),
        jnp.concatenate([embed[:, DH:], jnp.zeros((npad, DH), jnp.float32)]),
    ])
    embs = [embed]
    for _ in range(N_HOPS):
        agg2 = _hop(agg2, row, col, tr)
        embs.append(jnp.concatenate([agg2[0, :N_NODES], agg2[1, :N_NODES]], axis=1))
    return jnp.stack(embs, axis=1)
